# baseline (speedup 1.0000x reference)
import functools

import jax
import jax.numpy as jnp
from jax import lax
from jax.experimental import pallas as pl
from jax.experimental.pallas import tpu as pltpu
from jax.experimental.pallas import tpu_sc as plsc

BATCH = 4
SEQ = 4096
DIM = 1024
CS = 8


def kernel(input, weight):
    info = plsc.get_sparse_core_info()
    NC, NS, L = info.num_cores, info.num_subcores, info.num_lanes
    NW = NC * NS
    rows_per_w = SEQ // NW
    n_chunks = rows_per_w // CS
    mesh = plsc.VectorSubcoreMesh(core_axis_name="c", subcore_axis_name="s")

    scratch = (
        [pltpu.VMEM((BATCH, CS, DIM), jnp.float32) for _ in range(3)]
        + [pltpu.VMEM((CS, DIM), jnp.float32) for _ in range(3)]
        + [pltpu.SemaphoreType.DMA for _ in range(3)]
        + [pltpu.SemaphoreType.DMA for _ in range(3)]
        + [pltpu.SemaphoreType.DMA for _ in range(3)]
    )

    @functools.partial(
        pl.kernel,
        mesh=mesh,
        out_type=jax.ShapeDtypeStruct((BATCH, SEQ, DIM), jnp.float32),
        scratch_types=scratch,
    )
    def k(in_hbm, w_hbm, out_hbm, *sc):
        xs = sc[0:3]
        ws = sc[3:6]
        in_sems = sc[6:9]
        out_sems = sc[9:12]
        w_sems = sc[12:15]

        wid = lax.axis_index("s") * NC + lax.axis_index("c")
        base0 = wid * rows_per_w

        def issue_ins(c, g):
            base = base0 + c * CS
            for b in range(BATCH):
                pltpu.async_copy(
                    in_hbm.at[b, pl.ds(base, CS)], xs[g].at[b], in_sems[g]
                )

        def wait_ins(c, g):
            base = base0 + c * CS
            for b in range(BATCH):
                pltpu.make_async_copy(
                    in_hbm.at[b, pl.ds(base, CS)], xs[g].at[b], in_sems[g]
                ).wait()

        def issue_out(c, g, b):
            base = base0 + c * CS
            pltpu.async_copy(
                xs[g].at[b], out_hbm.at[b, pl.ds(base, CS)], out_sems[g]
            )

        def wait_outs(c, g):
            base = base0 + c * CS
            for b in range(BATCH):
                pltpu.make_async_copy(
                    xs[g].at[b], out_hbm.at[b, pl.ds(base, CS)], out_sems[g]
                ).wait()

        def issue_w(c, g):
            pltpu.async_copy(
                w_hbm.at[pl.ds(base0 + c * CS, CS)], ws[g], w_sems[g]
            )

        def wait_w(c, g):
            pltpu.make_async_copy(
                w_hbm.at[pl.ds(base0 + c * CS, CS)], ws[g], w_sems[g]
            ).wait()

        def chunk_step(c, g, do_drain, do_prefetch):
            wait_w(c, g)
            wait_ins(c, g)
            if do_drain:
                wait_outs(c - 1, (g + 2) % 3)
            if do_prefetch:
                issue_ins(c + 2, (g + 2) % 3)
                issue_w(c + 2, (g + 2) % 3)
            w_v = ws[g]
            x_v = xs[g]
            for b in range(BATCH):

                @plsc.parallel_loop(0, CS, step=1, unroll=1)
                def row_body(r, b=b):
                    for col in range(DIM // L):
                        sl = pl.ds(col * L, L)
                        plsc.addupdate(x_v.at[b, r, sl], w_v[r, sl])

                issue_out(c, g, b)

        issue_ins(0, 0)
        issue_ins(1, 1)
        issue_w(0, 0)
        issue_w(1, 1)

        chunk_step(0, 0, False, True)
        chunk_step(1, 1, True, True)

        def mid_body(t, _):
            cb = 2 + t * 3
            for u in range(3):
                chunk_step(cb + u, (2 + u) % 3, True, True)
            return 0

        lax.fori_loop(0, (n_chunks - 4) // 3, mid_body, 0)

        chunk_step(n_chunks - 2, (n_chunks - 2) % 3, True, False)
        chunk_step(n_chunks - 1, (n_chunks - 1) % 3, True, False)

        wait_outs(n_chunks - 1, (n_chunks - 1) % 3)

    return k(input, weight)


# speedup vs baseline: 1.0323x; 1.0323x over previous
import functools

import jax
import jax.numpy as jnp
from jax import lax
from jax.experimental import pallas as pl
from jax.experimental.pallas import tpu as pltpu
from jax.experimental.pallas import tpu_sc as plsc

BATCH = 4
SEQ = 4096
DIM = 1024
CS = 8


def kernel(input, weight):
    info = plsc.get_sparse_core_info()
    NC, NS, L = info.num_cores, info.num_subcores, info.num_lanes
    NW = NC * NS
    rows_per_w = SEQ // NW
    n_chunks = rows_per_w // CS
    mesh = plsc.VectorSubcoreMesh(core_axis_name="c", subcore_axis_name="s")

    scratch = (
        [pltpu.VMEM((BATCH, CS, DIM), jnp.float32) for _ in range(3)]
        + [pltpu.VMEM((CS, DIM), jnp.float32) for _ in range(3)]
        + [pltpu.SemaphoreType.DMA for _ in range(3)]
        + [pltpu.SemaphoreType.DMA for _ in range(3)]
        + [pltpu.SemaphoreType.DMA for _ in range(3)]
    )

    @functools.partial(
        pl.kernel,
        mesh=mesh,
        out_type=jax.ShapeDtypeStruct((BATCH, SEQ, DIM), jnp.float32),
        scratch_types=scratch,
    )
    def k(in_hbm, w_hbm, out_hbm, *sc):
        xs = sc[0:3]
        ws = sc[3:6]
        in_sems = sc[6:9]
        out_sems = sc[9:12]
        w_sems = sc[12:15]

        wid = lax.axis_index("s") * NC + lax.axis_index("c")
        base0 = wid * rows_per_w

        def issue_ins(c, g):
            base = base0 + c * CS
            for b in range(BATCH):
                pltpu.async_copy(
                    in_hbm.at[b, pl.ds(base, CS)], xs[g].at[b], in_sems[g]
                )

        def wait_ins(c, g):
            base = base0 + c * CS
            for b in range(BATCH):
                pltpu.make_async_copy(
                    in_hbm.at[b, pl.ds(base, CS)], xs[g].at[b], in_sems[g]
                ).wait()

        def issue_out(c, g, b):
            base = base0 + c * CS
            pltpu.async_copy(
                xs[g].at[b], out_hbm.at[b, pl.ds(base, CS)], out_sems[g]
            )

        def wait_outs(c, g):
            base = base0 + c * CS
            for b in range(BATCH):
                pltpu.make_async_copy(
                    xs[g].at[b], out_hbm.at[b, pl.ds(base, CS)], out_sems[g]
                ).wait()

        def issue_w(c, g):
            pltpu.async_copy(
                w_hbm.at[pl.ds(base0 + c * CS, CS)], ws[g], w_sems[g]
            )

        def wait_w(c, g):
            pltpu.make_async_copy(
                w_hbm.at[pl.ds(base0 + c * CS, CS)], ws[g], w_sems[g]
            ).wait()

        def chunk_step(c, g, do_drain, do_prefetch):
            wait_w(c, g)
            wait_ins(c, g)
            if do_drain:
                wait_outs(c - 1, (g + 2) % 3)
            if do_prefetch:
                issue_ins(c + 2, (g + 2) % 3)
                issue_w(c + 2, (g + 2) % 3)
            w_v = ws[g]
            x_v = xs[g]
            for b0 in (0, 2):

                @plsc.parallel_loop(0, CS, step=1, unroll=1)
                def row_body(r, b0=b0):
                    for col in range(DIM // L):
                        sl = pl.ds(col * L, L)
                        wvec = w_v[r, sl]
                        plsc.addupdate(x_v.at[b0, r, sl], wvec)
                        plsc.addupdate(x_v.at[b0 + 1, r, sl], wvec)

                issue_out(c, g, b0)
                issue_out(c, g, b0 + 1)

        issue_ins(0, 0)
        issue_ins(1, 1)
        issue_w(0, 0)
        issue_w(1, 1)

        chunk_step(0, 0, False, True)
        chunk_step(1, 1, True, True)

        def mid_body(t, _):
            cb = 2 + t * 3
            for u in range(3):
                chunk_step(cb + u, (2 + u) % 3, True, True)
            return 0

        lax.fori_loop(0, (n_chunks - 4) // 3, mid_body, 0)

        chunk_step(n_chunks - 2, (n_chunks - 2) % 3, True, False)
        chunk_step(n_chunks - 1, (n_chunks - 1) % 3, True, False)

        wait_outs(n_chunks - 1, (n_chunks - 1) % 3)

    return k(input, weight)
